# baseline (speedup 1.0000x reference)
import functools

import jax
import jax.numpy as jnp
from jax import lax
from jax.experimental import pallas as pl
from jax.experimental.pallas import tpu as pltpu
from jax.experimental.pallas import tpu_sc as plsc

VOCAB = 1000000
DIM = 64
B = 4096
N = 200
NC, NS = 2, 16
NW = NC * NS
NGRP = N // 8
NCB = B // 128
NTASK = NGRP * NCB
PER_W = NTASK // NW

_mesh = plsc.VectorSubcoreMesh(core_axis_name="c", subcore_axis_name="s")


@functools.partial(
    pl.kernel,
    out_type=jax.ShapeDtypeStruct((N, DIM, B), jnp.float32),
    mesh=_mesh,
    compiler_params=pltpu.CompilerParams(
        use_tc_tiling_on_sc=True,
        needs_layout_passes=False,
        disable_bounds_checks=True,
    ),
    scratch_types=[
        pltpu.VMEM((8, 128), jnp.int32),
        pltpu.VMEM((8, 128), jnp.int32),
        pltpu.VMEM((8, 128), jnp.int32),
        pltpu.VMEM((128, 129), jnp.float32),
        pltpu.VMEM((128, 129), jnp.float32),
        pltpu.VMEM((DIM, 128), jnp.float32),
        pltpu.VMEM((DIM, 128), jnp.float32),
        pltpu.SemaphoreType.DMA,
        pltpu.SemaphoreType.DMA,
        pltpu.SemaphoreType.DMA,
        pltpu.SemaphoreType.DMA,
        pltpu.SemaphoreType.DMA,
    ],
)
def _emb_lookup(idsT_hbm, tab_hbm, out_hbm, idt_v, pidx_v, hoff_v, rows0, rows1, t0, t1,
                isem, gsem0, gsem1, wsem0, wsem1):
    wid = lax.axis_index("s") * NC + lax.axis_index("c")
    rows = (rows0, rows1)
    tbufs = (t0, t1)
    gsems = (gsem0, gsem1)
    wsems = (wsem0, wsem1)


    rvs = tuple(lax.iota(jnp.int32, 16) + j0 for j0 in range(0, 128, 16))

    def start_gather(r, b):
        pltpu.async_copy(tab_hbm.at[pidx_v.at[r]], rows[b].at[:, pl.ds(0, 128)], gsems[b])

    def wait_gather(r, b):
        pltpu.make_async_copy(tab_hbm.at[pidx_v.at[r]],
                              rows[b].at[:, pl.ds(0, 128)], gsems[b]).wait()

    def transpose_block(r, b):
        tb = tbufs[b]
        rb = rows[b]
        hvs = tuple(hoff_v[r, pl.ds(j0 * 16, 16)] for j0 in range(8))

        @plsc.parallel_loop(0, DIM, step=1, unroll=8)
        def dbody(d):
            for j0 in range(8):
                vec = plsc.load_gather(rb, [rvs[j0], hvs[j0] + d])
                tb[d, pl.ds(j0 * 16, 16)] = vec

    def start_write(g, c, r, b):
        pltpu.async_copy(
            tbufs[b],
            out_hbm.at[g * 8 + r].at[:, pl.ds(c * 128, 128)],
            wsems[b])

    def wait_write(g, c, r, b):
        pltpu.make_async_copy(
            tbufs[b],
            out_hbm.at[g * 8 + r].at[:, pl.ds(c * 128, 128)],
            wsems[b]).wait()

    def run(t, carry):
        g = t // NCB
        c = t % NCB
        pltpu.async_copy(
            idsT_hbm.at[pl.ds(g * 8, 8), pl.ds(c * 128, 128)], idt_v,
            isem).wait()
        for r in range(8):
            for j0 in range(0, 128, 16):
                v = idt_v[r, pl.ds(j0, 16)]
                pidx_v[r, pl.ds(j0, 16)] = v >> 1
                hoff_v[r, pl.ds(j0, 16)] = (v & 1) * DIM
        start_gather(0, 0)
        for r in range(8):
            b = r % 2
            if r + 1 < 8:
                start_gather(r + 1, 1 - b)
            wait_gather(r, b)
            if r >= 2:
                wait_write(g, c, r - 2, b)
            transpose_block(r, b)
            start_write(g, c, r, b)
        wait_write(g, c, 6, 0)
        wait_write(g, c, 7, 1)
        return carry

    lax.fori_loop(wid * PER_W, (wid + 1) * PER_W, run, 0)


def kernel(input_ids, table):
    tab2 = table.reshape(VOCAB // 2, 2 * DIM)
    ids_t = input_ids.T
    out_t = _emb_lookup(ids_t, tab2)
    return jnp.transpose(out_t, (2, 0, 1))


# speedup vs baseline: 1.6717x; 1.6717x over previous
import functools

import jax
import jax.numpy as jnp
from jax import lax
from jax.experimental import pallas as pl
from jax.experimental.pallas import tpu as pltpu
from jax.experimental.pallas import tpu_sc as plsc

VOCAB = 1000000
DIM = 64
B = 4096
N = 200
TOT = B * N
NC, NS = 2, 16
NW = NC * NS
PER_W = TOT // NW
CHUNK = 400
NCHUNK = PER_W // CHUNK
NBUF = 4

_mesh = plsc.VectorSubcoreMesh(core_axis_name="c", subcore_axis_name="s")


@functools.partial(
    pl.kernel,
    out_type=jax.ShapeDtypeStruct((TOT, 2 * DIM), jnp.float32),
    mesh=_mesh,
    compiler_params=pltpu.CompilerParams(use_tc_tiling_on_sc=False),
    scratch_types=[
        pltpu.VMEM((PER_W,), jnp.int32),
        [pltpu.VMEM((CHUNK, DIM), jnp.float32) for _ in range(NBUF)],
        [pltpu.SemaphoreType.DMA for _ in range(NBUF)],
        [pltpu.SemaphoreType.DMA for _ in range(NBUF)],
        pltpu.SemaphoreType.DMA,
    ],
)
def _emb_lookup(idx_hbm, table_hbm, out_hbm, idx_v, rows, gsem, wsem, isem):
    wid = lax.axis_index("s") * NC + lax.axis_index("c")
    base = wid * PER_W

    pltpu.async_copy(idx_hbm.at[pl.ds(base, PER_W)], idx_v, isem).wait()

    def start_gather(k, b):
        pltpu.async_copy(
            table_hbm.at[idx_v.at[pl.ds(k * CHUNK, CHUNK)]], rows[b], gsem[b])

    def wait_gather(k, b):
        pltpu.make_async_copy(
            table_hbm.at[idx_v.at[pl.ds(k * CHUNK, CHUNK)]], rows[b],
            gsem[b]).wait()

    def start_write(k, b):
        pltpu.async_copy(rows[b],
                         out_hbm.at[pl.ds(base + k * CHUNK, CHUNK),
                                    pl.ds(0, DIM)],
                         wsem[b])

    def wait_write(k, b):
        pltpu.make_async_copy(rows[b],
                              out_hbm.at[pl.ds(base + k * CHUNK, CHUNK),
                                         pl.ds(0, DIM)],
                              wsem[b]).wait()

    start_gather(0, 0)
    start_gather(1, 1)
    start_gather(2, 2)
    wait_gather(0, 0)
    start_write(0, 0)
    start_gather(3, 3)
    wait_gather(1, 1)
    start_write(1, 1)

    STEADY_END = 2 + ((NCHUNK - 2 - 2) // NBUF) * NBUF

    def body(g, carry):
        k0 = 2 + g * NBUF
        for j in range(NBUF):
            k = k0 + j
            b = (2 + j) % NBUF
            b2 = j % NBUF
            wait_write(k - 2, b2)
            start_gather(k + 2, b2)
            wait_gather(k, b)
            start_write(k, b)
        return carry

    lax.fori_loop(0, (STEADY_END - 2) // NBUF, body, 0)

    for k in range(STEADY_END, NCHUNK):
        b = k % NBUF
        if k + 2 < NCHUNK:
            wait_write(k - 2, (k + 2) % NBUF)
            start_gather(k + 2, (k + 2) % NBUF)
        wait_gather(k, b)
        start_write(k, b)

    for k in range(NCHUNK - NBUF, NCHUNK):
        wait_write(k, k % NBUF)


def kernel(input_ids, table):
    flat = input_ids.reshape(TOT).astype(jnp.int32)
    out = _emb_lookup(flat, table)
    return out.reshape(B, N, 2 * DIM)[:, :, :DIM]
